# baseline (speedup 1.0000x reference)
import functools

import jax
import jax.numpy as jnp
from jax import lax
from jax.experimental import pallas as pl
from jax.experimental.pallas import tpu as pltpu
from jax.experimental.pallas import tpu_sc as plsc

BATCH = 4096
SEQ = 200
DIM = 64
PAD = 128
NUM_WORKERS = 32
CHUNK = 128
NB = BATCH // CHUNK
N_PAIRS = SEQ // 2


def _make_gather():
    mesh = plsc.VectorSubcoreMesh(core_axis_name="c", subcore_axis_name="s")

    @functools.partial(
        pl.kernel,
        mesh=mesh,
        out_type=jax.ShapeDtypeStruct((SEQ, 8, NB, 8 * CHUNK), jnp.float32),
        scratch_types=[
            pltpu.VMEM((SEQ * CHUNK,), jnp.int32),
            pltpu.VMEM((2 * CHUNK, PAD), jnp.float32),
            pltpu.VMEM((2 * CHUNK, PAD), jnp.float32),
            pltpu.VMEM((8, 8 * CHUNK), jnp.float32),
            pltpu.VMEM((8, 8 * CHUNK), jnp.float32),
            pltpu.SemaphoreType.DMA,
            pltpu.SemaphoreType.DMA,
            pltpu.SemaphoreType.DMA,
            pltpu.SemaphoreType.DMA,
        ],
        compiler_params=pltpu.CompilerParams(needs_layout_passes=False),
    )
    def gather_kernel(idx_hbm, table_hbm, out_hbm, idx_v, r_a, r_b,
                      t0, t1, g_a, g_b, o0, o1):
        w = lax.axis_index("s") * 2 + lax.axis_index("c")
        pltpu.sync_copy(idx_hbm.at[w], idx_v)

        tbufs = [t0, t1]
        osems = [o0, o1]

        lane = lax.iota(jnp.int32, 16)
        rot = [jnp.bitwise_and(lane + j, 15) for j in range(16)]

        def fire(u, rbuf, sem):
            off = pl.multiple_of(u * (2 * CHUNK), 2 * CHUNK)
            pltpu.async_copy(
                table_hbm.at[idx_v.at[pl.ds(off, 2 * CHUNK)]], rbuf, sem)

        def gather_wait(rbuf, sem):
            pltpu.make_async_copy(
                table_hbm.at[idx_v.at[pl.ds(0, 2 * CHUNK)]], rbuf, sem).wait()

        def transpose(rbuf, base, tbuf):
            @plsc.parallel_loop(0, 8, unroll=2)
            def gi_body(gi):
                cvec = lane + (16 * gi + base)
                dvec = lane + 16 * gi
                for eb in range(4):
                    for j in range(16):
                        v = plsc.load_gather(rbuf, [cvec, rot[j] + eb * 16])
                        te = jnp.right_shift(rot[j], 3) + eb * 2
                        fl = jnp.left_shift(jnp.bitwise_and(rot[j], 7), 7)
                        plsc.store_scatter(tbuf, [te, fl + dvec], v)

        def out_start(s, tbuf, sem):
            pltpu.async_copy(tbuf, out_hbm.at[s, :, w], sem)

        def out_wait(tbuf, sem):
            pltpu.make_async_copy(tbuf, out_hbm.at[0, :, w], sem).wait()

        n_streams = SEQ // 2

        def step(u_proc, u_fire, rbuf, gsem):
            gather_wait(rbuf, gsem)
            for h in range(2):
                out_wait(tbufs[h], osems[h])
                transpose(rbuf, h * CHUNK, tbufs[h])
                out_start(2 * u_proc + h, tbufs[h], osems[h])
            fire(u_fire, rbuf, gsem)

        fire(0, r_a, g_a)
        fire(1, r_b, g_b)
        out_start(0, t0, o0)
        out_start(1, t1, o1)

        def body(p, carry):
            u0 = 2 * p
            step(u0, jnp.minimum(u0 + 2, n_streams - 1), r_a, g_a)
            step(u0 + 1, jnp.minimum(u0 + 3, n_streams - 1), r_b, g_b)
            return carry

        lax.fori_loop(0, n_streams // 2, body, 0)
        gather_wait(r_a, g_a)
        gather_wait(r_b, g_b)
        out_wait(t0, o0)
        out_wait(t1, o1)

    return gather_kernel


_gather = _make_gather()


def kernel(word_inputs, word_seq_lengths, char_inputs, char_seq_lengths,
           char_seq_recover, word_embeddings):
    idx = (word_inputs.T.astype(jnp.int32).reshape(SEQ, NB, CHUNK)
           .transpose(1, 0, 2).reshape(NB, SEQ * CHUNK))
    table = jnp.pad(word_embeddings, ((0, 0), (0, PAD - DIM)))
    x = _gather(idx, table).reshape(SEQ, 8, NB, 8, CHUNK)
    return x.transpose(2, 4, 0, 1, 3).reshape(BATCH, SEQ, DIM)


# speedup vs baseline: 1.3456x; 1.3456x over previous
import functools

import jax
import jax.numpy as jnp
from jax import lax
from jax.experimental import pallas as pl
from jax.experimental.pallas import tpu as pltpu
from jax.experimental.pallas import tpu_sc as plsc

BATCH = 4096
SEQ = 200
DIM = 64
PAD = 128
NUM_WORKERS = 32
CHUNK = 128
NB = BATCH // CHUNK
N_PAIRS = SEQ // 2


def _make_gather():
    mesh = plsc.VectorSubcoreMesh(core_axis_name="c", subcore_axis_name="s")

    @functools.partial(
        pl.kernel,
        mesh=mesh,
        out_type=jax.ShapeDtypeStruct((SEQ, 8, NB, 8, CHUNK), jnp.float32),
        scratch_types=[
            pltpu.VMEM((SEQ * CHUNK,), jnp.int32),
            pltpu.VMEM((2 * CHUNK, PAD), jnp.float32),
            pltpu.VMEM((2 * CHUNK, PAD), jnp.float32),
            pltpu.VMEM((8, 8, CHUNK), jnp.float32),
            pltpu.VMEM((8, 8, CHUNK), jnp.float32),
            pltpu.SemaphoreType.DMA,
            pltpu.SemaphoreType.DMA,
            pltpu.SemaphoreType.DMA,
            pltpu.SemaphoreType.DMA,
        ],
        compiler_params=pltpu.CompilerParams(needs_layout_passes=False),
    )
    def gather_kernel(idx_hbm, table_hbm, out_hbm, idx_v, r_a, r_b,
                      t0, t1, g_a, g_b, o0, o1):
        w = lax.axis_index("s") * 2 + lax.axis_index("c")
        pltpu.sync_copy(idx_hbm.at[w], idx_v)

        tbufs = [t0, t1]
        osems = [o0, o1]

        lane = lax.iota(jnp.int32, 16)
        rot = [jnp.bitwise_and(lane + j, 15) for j in range(16)]

        def fire(u, rbuf, sem):
            off = pl.multiple_of(u * (2 * CHUNK), 2 * CHUNK)
            pltpu.async_copy(
                table_hbm.at[idx_v.at[pl.ds(off, 2 * CHUNK)]], rbuf, sem)

        def gather_wait(rbuf, sem):
            pltpu.make_async_copy(
                table_hbm.at[idx_v.at[pl.ds(0, 2 * CHUNK)]], rbuf, sem).wait()

        def transpose(rbuf, base, tbuf):
            @plsc.parallel_loop(0, 8)
            def gi_body(gi):
                cvec = lane + (16 * gi + base)
                dvec = lane + 16 * gi
                for eb in range(4):
                    for j in range(16):
                        v = plsc.load_gather(rbuf, [cvec, rot[j] + eb * 16])
                        te = jnp.right_shift(rot[j], 3) + eb * 2
                        rv = jnp.bitwise_and(rot[j], 7)
                        plsc.store_scatter(tbuf, [te, rv, dvec], v)

        def out_start(s, tbuf, sem):
            pltpu.async_copy(tbuf, out_hbm.at[s, :, w], sem)

        def out_wait(tbuf, sem):
            pltpu.make_async_copy(tbuf, out_hbm.at[0, :, w], sem).wait()

        n_streams = SEQ // 2

        def step(u_proc, u_fire, rbuf, gsem):
            gather_wait(rbuf, gsem)
            for h in range(2):
                out_wait(tbufs[h], osems[h])
                transpose(rbuf, h * CHUNK, tbufs[h])
                out_start(2 * u_proc + h, tbufs[h], osems[h])
            fire(u_fire, rbuf, gsem)

        fire(0, r_a, g_a)
        fire(1, r_b, g_b)
        out_start(0, t0, o0)
        out_start(1, t1, o1)

        def body(p, carry):
            u0 = 2 * p
            step(u0, jnp.minimum(u0 + 2, n_streams - 1), r_a, g_a)
            step(u0 + 1, jnp.minimum(u0 + 3, n_streams - 1), r_b, g_b)
            return carry

        lax.fori_loop(0, n_streams // 2, body, 0)
        gather_wait(r_a, g_a)
        gather_wait(r_b, g_b)
        out_wait(t0, o0)
        out_wait(t1, o1)

    return gather_kernel


_gather = _make_gather()


def kernel(word_inputs, word_seq_lengths, char_inputs, char_seq_lengths,
           char_seq_recover, word_embeddings):
    idx = (word_inputs.T.astype(jnp.int32).reshape(SEQ, NB, CHUNK)
           .transpose(1, 0, 2).reshape(NB, SEQ * CHUNK))
    table = jnp.pad(word_embeddings, ((0, 0), (0, PAD - DIM)))
    x = _gather(idx, table)
    return x.transpose(2, 4, 0, 1, 3).reshape(BATCH, SEQ, DIM)
